# Initial kernel scaffold; baseline (speedup 1.0000x reference)
import jax
import jax.numpy as jnp
from jax.experimental import pallas as pl


def kernel(inputs, table):
    raise NotImplementedError("write your pallas kernel here")


# speedup vs baseline: 4.2650x; 4.2650x over previous
import functools

import jax
import jax.numpy as jnp
from jax.experimental import pallas as pl
from jax.experimental.pallas import tpu as pltpu
from jax.experimental.pallas import tpu_sc as plsc

_W = 128


def kernel(inputs, table):
    B, T = inputs.shape
    V, D = table.shape
    N = B * T
    idx = inputs.reshape(1, N).astype(jnp.int32)

    mesh = plsc.VectorSubcoreMesh(
        core_axis_name="core", subcore_axis_name="subcore"
    )

    @functools.partial(
        pl.kernel,
        out_type=jax.ShapeDtypeStruct((N, D), table.dtype),
        mesh=mesh,
        compiler_params=pltpu.CompilerParams(use_tc_tiling_on_sc=False),
    )
    def gather_kernel(table_hbm, idx_hbm, out_hbm):
        def body(i_vmem, o_vmem):
            pltpu.sync_copy(table_hbm.at[i_vmem.at[0]], o_vmem)

        pltpu.emit_pipeline(
            body,
            grid=(N // _W,),
            in_specs=[pl.BlockSpec((1, _W), index_map=lambda i: (0, i))],
            out_specs=[pl.BlockSpec((_W, D), index_map=lambda i: (i, 0))],
            core_axis_name=("core", "subcore"),
            dimension_semantics=(pltpu.PARALLEL,),
        )(idx_hbm, out_hbm)

    out = gather_kernel(table, idx)
    return out.reshape(B, T, D)


# speedup vs baseline: 4.6006x; 1.0787x over previous
import functools

import jax
import jax.numpy as jnp
from jax.experimental import pallas as pl
from jax.experimental.pallas import tpu as pltpu
from jax.experimental.pallas import tpu_sc as plsc

_W = 512


def kernel(inputs, table):
    B, T = inputs.shape
    V, D = table.shape
    N = B * T
    idx = inputs.reshape(1, N).astype(jnp.int32)

    mesh = plsc.VectorSubcoreMesh(
        core_axis_name="core", subcore_axis_name="subcore"
    )

    @functools.partial(
        pl.kernel,
        out_type=jax.ShapeDtypeStruct((N, D), table.dtype),
        mesh=mesh,
        compiler_params=pltpu.CompilerParams(use_tc_tiling_on_sc=False),
    )
    def gather_kernel(table_hbm, idx_hbm, out_hbm):
        def body(i_vmem, o_vmem):
            pltpu.sync_copy(table_hbm.at[i_vmem.at[0]], o_vmem)

        pltpu.emit_pipeline(
            body,
            grid=(N // _W,),
            in_specs=[pl.BlockSpec((1, _W), index_map=lambda i: (0, i))],
            out_specs=[pl.BlockSpec((_W, D), index_map=lambda i: (i, 0))],
            core_axis_name=("core", "subcore"),
            dimension_semantics=(pltpu.PARALLEL,),
        )(idx_hbm, out_hbm)

    out = gather_kernel(table, idx)
    return out.reshape(B, T, D)


# speedup vs baseline: 4.6218x; 1.0046x over previous
import functools

import jax
import jax.numpy as jnp
from jax.experimental import pallas as pl
from jax.experimental.pallas import tpu as pltpu
from jax.experimental.pallas import tpu_sc as plsc

_W = 800


def kernel(inputs, table):
    B, T = inputs.shape
    V, D = table.shape
    N = B * T
    idx = inputs.reshape(1, N).astype(jnp.int32)

    mesh = plsc.VectorSubcoreMesh(
        core_axis_name="core", subcore_axis_name="subcore"
    )

    @functools.partial(
        pl.kernel,
        out_type=jax.ShapeDtypeStruct((N, D), table.dtype),
        mesh=mesh,
        compiler_params=pltpu.CompilerParams(use_tc_tiling_on_sc=False),
    )
    def gather_kernel(table_hbm, idx_hbm, out_hbm):
        def body(i_vmem, o_vmem):
            pltpu.sync_copy(table_hbm.at[i_vmem.at[0]], o_vmem)

        pltpu.emit_pipeline(
            body,
            grid=(N // _W,),
            in_specs=[pl.BlockSpec((1, _W), index_map=lambda i: (0, i))],
            out_specs=[pl.BlockSpec((_W, D), index_map=lambda i: (i, 0))],
            core_axis_name=("core", "subcore"),
            dimension_semantics=(pltpu.PARALLEL,),
        )(idx_hbm, out_hbm)

    out = gather_kernel(table, idx)
    return out.reshape(B, T, D)


# speedup vs baseline: 5.6775x; 1.2284x over previous
import functools

import jax
import jax.numpy as jnp
from jax import lax
from jax.experimental import pallas as pl
from jax.experimental.pallas import tpu as pltpu
from jax.experimental.pallas import tpu_sc as plsc

_NBUF = 4


def kernel(inputs, table):
    B, T = inputs.shape
    V, D = table.shape
    idx = inputs.astype(jnp.int32)
    table_ext = jnp.concatenate(
        [table, jnp.zeros((V, 128 - D), table.dtype)], axis=1
    )

    info = plsc.get_sparse_core_info()
    nw = info.num_cores * info.num_subcores
    b_per_w = B // nw

    mesh = plsc.VectorSubcoreMesh(core_axis_name="c", subcore_axis_name="s")

    @functools.partial(
        pl.kernel,
        out_type=jax.ShapeDtypeStruct((B, T, 128), table.dtype),
        mesh=mesh,
        scratch_types=[
            pltpu.VMEM((b_per_w, T), jnp.int32),
            pltpu.VMEM((_NBUF, T, 128), jnp.float32),
            pltpu.SemaphoreType.DMA,
            pltpu.SemaphoreType.DMA,
        ],
    )
    def gather_kernel(tab_hbm, idx_hbm, out_hbm, idx_v, rows_v, gsem, osem):
        wid = lax.axis_index("s") * info.num_cores + lax.axis_index("c")
        b0 = wid * b_per_w
        pltpu.sync_copy(idx_hbm.at[pl.ds(b0, b_per_w)], idx_v)

        @pl.loop(0, b_per_w, step=_NBUF)
        def _(j):
            for k in range(_NBUF):
                pltpu.make_async_copy(
                    tab_hbm.at[idx_v.at[j + k]], rows_v.at[k], gsem
                ).start()
            for k in range(_NBUF):
                pltpu.make_async_copy(
                    tab_hbm.at[idx_v.at[j + k]], rows_v.at[k], gsem
                ).wait()
                pltpu.make_async_copy(
                    rows_v.at[k], out_hbm.at[b0 + j + k], osem
                ).start()
            for k in range(_NBUF):
                pltpu.make_async_copy(
                    rows_v.at[k], out_hbm.at[b0 + j + k], osem
                ).wait()

    return gather_kernel(table_ext, idx)[:, :, :D]


# speedup vs baseline: 6.1007x; 1.0745x over previous
import functools

import jax
import jax.numpy as jnp
from jax import lax
from jax.experimental import pallas as pl
from jax.experimental.pallas import tpu as pltpu
from jax.experimental.pallas import tpu_sc as plsc

_NBUF = 4


def kernel(inputs, table):
    B, T = inputs.shape
    V, D = table.shape
    idx = inputs.astype(jnp.int32)
    table_ext = jnp.concatenate(
        [table, jnp.zeros((V, 128 - D), table.dtype)], axis=1
    )

    info = plsc.get_sparse_core_info()
    nw = info.num_cores * info.num_subcores
    b_per_w = B // nw
    tp = T // 2

    mesh = plsc.VectorSubcoreMesh(core_axis_name="c", subcore_axis_name="s")

    @functools.partial(
        pl.kernel,
        out_type=jax.ShapeDtypeStruct((B, tp, 2 * D), table.dtype),
        mesh=mesh,
        scratch_types=[
            pltpu.VMEM((b_per_w, T), jnp.int32),
            pltpu.VMEM((_NBUF, T, 128), jnp.float32),
            pltpu.VMEM((_NBUF, tp, 128), jnp.float32),
            pltpu.SemaphoreType.DMA,
            pltpu.SemaphoreType.DMA,
        ],
    )
    def gather_kernel(
        tab_hbm, idx_hbm, out_hbm, idx_v, rows_v, pack_v, gsem, osem
    ):
        wid = lax.axis_index("s") * info.num_cores + lax.axis_index("c")
        b0 = wid * b_per_w
        pltpu.sync_copy(idx_hbm.at[pl.ds(b0, b_per_w)], idx_v)

        for k in range(_NBUF):
            pltpu.make_async_copy(
                tab_hbm.at[idx_v.at[k]], rows_v.at[k], gsem
            ).start()

        @pl.loop(0, b_per_w, step=_NBUF)
        def _(j):
            for k in range(_NBUF):
                pltpu.make_async_copy(
                    tab_hbm.at[idx_v.at[j + k]], rows_v.at[k], gsem
                ).wait()
                for r in range(T):
                    p, h = r // 2, (r % 2) * D
                    for c in range(0, D, 16):
                        pack_v[k, p, pl.ds(h + c, 16)] = rows_v[
                            k, r, pl.ds(c, 16)
                        ]

                @pl.when(j + _NBUF < b_per_w)
                def _():
                    pltpu.make_async_copy(
                        tab_hbm.at[idx_v.at[j + _NBUF + k]], rows_v.at[k], gsem
                    ).start()

                pltpu.make_async_copy(
                    pack_v.at[k], out_hbm.at[b0 + j + k], osem
                ).start()
            for k in range(_NBUF):
                pltpu.make_async_copy(
                    pack_v.at[k], out_hbm.at[b0 + j + k], osem
                ).wait()

    return gather_kernel(table_ext, idx).reshape(B, T, D)
